# baseline (speedup 1.0000x reference)
import functools

import jax
import jax.numpy as jnp
from jax.experimental import pallas as pl
from jax.experimental.pallas import tpu as pltpu

_H = 12
_DK = 64
_NG = _H // 3
_NRED = 8
_NEMIT = 4
_NROWS = _H + 2 * _NG


def _fused_body(values_ref, wv_ref, bv_ref, wo_ref, bo_ref, out_ref,
                acc_ref, rows_ref, *, inv_l, d, dk, r1, off1, r2, off2,
                rows_per_group, emit_rows):
    i = pl.program_id(0)

    @pl.when(i < _NRED)
    def _reduce():
        psum = jnp.sum(values_ref[...], axis=0, keepdims=True)
        prev = jnp.where(i == 0, jnp.zeros_like(psum), acc_ref[...])
        acc_ref[...] = prev + psum

    @pl.when(i == _NRED - 1)
    def _build_rows():
        colmean = acc_ref[...] * inv_l
        meanv = jax.lax.dot_general(
            colmean, wv_ref[...], (((1,), (1,)), ((), ())),
            preferred_element_type=jnp.float32) + bv_ref[...]
        heads = jnp.concatenate(
            [meanv[:, h * dk:(h + 1) * dk] for h in range(_H)], axis=0)
        tiled = jnp.concatenate([heads] * (d // dk), axis=1)
        gi = jax.lax.broadcasted_iota(jnp.int32, (_NG, _H), 0)
        hi = jax.lax.broadcasted_iota(jnp.int32, (_NG, _H), 1)
        sa = (hi == 3 * gi).astype(jnp.float32)
        sb = (hi == 3 * gi + 1).astype(jnp.float32)
        sc = (hi == 3 * gi + 2).astype(jnp.float32)
        dn = (((1,), (0,)), ((), ()))
        arows = jax.lax.dot_general(sa, tiled, dn,
                                    preferred_element_type=jnp.float32)
        brows = jax.lax.dot_general(sb, tiled, dn,
                                    preferred_element_type=jnp.float32)
        crows = jax.lax.dot_general(sc, tiled, dn,
                                    preferred_element_type=jnp.float32)
        lane = jax.lax.broadcasted_iota(jnp.int32, (_NG, d), 1)
        mab = jnp.where(lane < off1, arows, brows)
        mbc = jnp.where(lane < off2, brows, crows)
        ctx20 = jnp.concatenate([tiled, mab, mbc], axis=0)
        rows_ref[0:_NROWS, :] = jax.lax.dot_general(
            ctx20, wo_ref[...], (((1,), (1,)), ((), ())),
            preferred_element_type=jnp.float32) + bo_ref[...]

    @pl.when(i >= _NRED)
    def _emit():
        j = i - _NRED
        rows = rows_ref[0:_NROWS, :]
        base = j * emit_rows
        rid = jax.lax.broadcasted_iota(jnp.int32, (emit_rows, _NROWS), 0) + base
        kid = jax.lax.broadcasted_iota(jnp.int32, (emit_rows, _NROWS), 1)
        grp = rid // rows_per_group
        loc = rid - grp * rows_per_group
        rtype = jnp.where(
            loc < r1, 3 * grp,
            jnp.where(loc == r1, _H + grp,
                      jnp.where(loc < r2, 3 * grp + 1,
                                jnp.where(loc == r2, _H + _NG + grp,
                                          3 * grp + 2))))
        sel = (kid == rtype).astype(jnp.float32)
        out_ref[...] = jax.lax.dot_general(
            sel, rows, (((1,), (0,)), ((), ())),
            preferred_element_type=jnp.float32)


def kernel(queries, keys, values, Wq, bq, Wk, bk, Wv, bv, Wo, bo):
    b, l, d = values.shape
    dk = _DK
    vals2d = values.reshape(b * l, d)
    red_blk = (b * l) // _NRED
    emit_rows = (b * l) // _NEMIT

    stream = l * dk
    rows_per_group = 3 * stream // d
    r1, off1 = stream // d, stream % d
    r2, off2 = (2 * stream) // d, (2 * stream) % d

    out2d = pl.pallas_call(
        functools.partial(_fused_body, inv_l=1.0 / (b * l), d=d, dk=dk,
                          r1=r1, off1=off1, r2=r2, off2=off2,
                          rows_per_group=rows_per_group, emit_rows=emit_rows),
        grid=(_NRED + _NEMIT,),
        in_specs=[
            pl.BlockSpec((red_blk, d),
                         lambda i: (jnp.minimum(i, _NRED - 1), 0)),
            pl.BlockSpec((d, d), lambda i: (0, 0)),
            pl.BlockSpec((1, d), lambda i: (0, 0)),
            pl.BlockSpec((d, d), lambda i: (0, 0)),
            pl.BlockSpec((1, d), lambda i: (0, 0)),
        ],
        out_specs=pl.BlockSpec((emit_rows, d),
                               lambda i: (jnp.maximum(i, _NRED) - _NRED, 0)),
        out_shape=jax.ShapeDtypeStruct((b * l, d), jnp.float32),
        scratch_shapes=[pltpu.VMEM((1, d), jnp.float32),
                        pltpu.VMEM((24, d), jnp.float32)],
    )(vals2d, Wv, bv.reshape(1, d), Wo, bo.reshape(1, d))

    return out2d.reshape(b, l, d)


# speedup vs baseline: 1.1194x; 1.1194x over previous
import functools

import jax
import jax.numpy as jnp
from jax.experimental import pallas as pl
from jax.experimental.pallas import tpu as pltpu

_H = 12
_DK = 64
_NG = _H // 3
_NRED = 4
_NEMIT = 4
_NROWS = _H + 2 * _NG


def _fused_body(values_ref, wv_ref, bv_ref, wo_ref, bo_ref, out_ref,
                acc_ref, rows_ref, wv_s, wo_s, *, inv_l, d, dk, r1, off1,
                r2, off2, rows_per_group, emit_rows):
    i = pl.program_id(0)

    @pl.when(i < _NRED)
    def _reduce():
        psum = jnp.sum(values_ref[...], axis=0, keepdims=True)
        prev = jnp.where(i == 0, jnp.zeros_like(psum), acc_ref[...])
        acc_ref[...] = prev + psum
        wblk = d // _NRED
        wv_s[pl.ds(i * wblk, wblk), :] = wv_ref[...]
        wo_s[pl.ds(i * wblk, wblk), :] = wo_ref[...]

    @pl.when(i == _NRED - 1)
    def _build_rows():
        colmean = acc_ref[...] * inv_l
        meanv = jax.lax.dot_general(
            colmean, wv_s[...], (((1,), (1,)), ((), ())),
            preferred_element_type=jnp.float32) + bv_ref[...]
        heads = jnp.concatenate(
            [meanv[:, h * dk:(h + 1) * dk] for h in range(_H)], axis=0)
        tiled = jnp.concatenate([heads] * (d // dk), axis=1)
        gi = jax.lax.broadcasted_iota(jnp.int32, (_NG, _H), 0)
        hi = jax.lax.broadcasted_iota(jnp.int32, (_NG, _H), 1)
        sa = (hi == 3 * gi).astype(jnp.float32)
        sb = (hi == 3 * gi + 1).astype(jnp.float32)
        sc = (hi == 3 * gi + 2).astype(jnp.float32)
        dn = (((1,), (0,)), ((), ()))
        arows = jax.lax.dot_general(sa, tiled, dn,
                                    preferred_element_type=jnp.float32)
        brows = jax.lax.dot_general(sb, tiled, dn,
                                    preferred_element_type=jnp.float32)
        crows = jax.lax.dot_general(sc, tiled, dn,
                                    preferred_element_type=jnp.float32)
        lane = jax.lax.broadcasted_iota(jnp.int32, (_NG, d), 1)
        mab = jnp.where(lane < off1, arows, brows)
        mbc = jnp.where(lane < off2, brows, crows)
        ctx20 = jnp.concatenate([tiled, mab, mbc], axis=0)
        rows_ref[0:_NROWS, :] = jax.lax.dot_general(
            ctx20, wo_s[...], (((1,), (1,)), ((), ())),
            preferred_element_type=jnp.float32) + bo_ref[...]

    @pl.when(i >= _NRED)
    def _emit():
        j = i - _NRED
        rows = rows_ref[0:_NROWS, :]
        base = j * emit_rows
        rid = jax.lax.broadcasted_iota(jnp.int32, (emit_rows, _NROWS), 0) + base
        kid = jax.lax.broadcasted_iota(jnp.int32, (emit_rows, _NROWS), 1)
        grp = rid // rows_per_group
        loc = rid - grp * rows_per_group
        rtype = jnp.where(
            loc < r1, 3 * grp,
            jnp.where(loc == r1, _H + grp,
                      jnp.where(loc < r2, 3 * grp + 1,
                                jnp.where(loc == r2, _H + _NG + grp,
                                          3 * grp + 2))))
        sel = (kid == rtype).astype(jnp.float32)
        out_ref[...] = jax.lax.dot_general(
            sel, rows, (((1,), (0,)), ((), ())),
            preferred_element_type=jnp.float32)


def kernel(queries, keys, values, Wq, bq, Wk, bk, Wv, bv, Wo, bo):
    b, l, d = values.shape
    dk = _DK
    vals2d = values.reshape(b * l, d)
    red_blk = (b * l) // _NRED
    emit_rows = (b * l) // _NEMIT

    stream = l * dk
    rows_per_group = 3 * stream // d
    r1, off1 = stream // d, stream % d
    r2, off2 = (2 * stream) // d, (2 * stream) % d

    out2d = pl.pallas_call(
        functools.partial(_fused_body, inv_l=1.0 / (b * l), d=d, dk=dk,
                          r1=r1, off1=off1, r2=r2, off2=off2,
                          rows_per_group=rows_per_group, emit_rows=emit_rows),
        grid=(_NRED + _NEMIT,),
        in_specs=[
            pl.BlockSpec((red_blk, d),
                         lambda i: (jnp.minimum(i, _NRED - 1), 0)),
            pl.BlockSpec((d // _NRED, d),
                         lambda i: (jnp.minimum(i, _NRED - 1), 0)),
            pl.BlockSpec((1, d), lambda i: (0, 0)),
            pl.BlockSpec((d // _NRED, d),
                         lambda i: (jnp.minimum(i, _NRED - 1), 0)),
            pl.BlockSpec((1, d), lambda i: (0, 0)),
        ],
        out_specs=pl.BlockSpec((emit_rows, d),
                               lambda i: (jnp.maximum(i, _NRED) - _NRED, 0)),
        out_shape=jax.ShapeDtypeStruct((b * l, d), jnp.float32),
        scratch_shapes=[pltpu.VMEM((1, d), jnp.float32),
                        pltpu.VMEM((24, d), jnp.float32),
                        pltpu.VMEM((d, d), jnp.float32),
                        pltpu.VMEM((d, d), jnp.float32)],
    )(vals2d, Wv, bv.reshape(1, d), Wo, bo.reshape(1, d))

    return out2d.reshape(b, l, d)


# speedup vs baseline: 1.1311x; 1.0105x over previous
import functools

import jax
import jax.numpy as jnp
from jax.experimental import pallas as pl
from jax.experimental.pallas import tpu as pltpu

_H = 12
_DK = 64
_NG = _H // 3
_NRED = 4
_NEMIT = 4
_NROWS = _H + 2 * _NG


def _fused_body(values_ref, wv_ref, bv_ref, wo_ref, bo_ref, out_ref,
                acc_ref, rows_ref, wv_s, wo_s, sem_v, sem_o, *, inv_l, d,
                dk, r1, off1, r2, off2, rows_per_group, emit_rows):
    i = pl.program_id(0)

    @pl.when(i == 0)
    def _start_weight_dma():
        pltpu.make_async_copy(wv_ref, wv_s, sem_v).start()
        pltpu.make_async_copy(wo_ref, wo_s, sem_o).start()

    @pl.when(i < _NRED)
    def _reduce():
        psum = jnp.sum(values_ref[...], axis=0, keepdims=True)
        prev = jnp.where(i == 0, jnp.zeros_like(psum), acc_ref[...])
        acc_ref[...] = prev + psum

    @pl.when(i == _NRED - 1)
    def _build_rows():
        pltpu.make_async_copy(wv_ref, wv_s, sem_v).wait()
        pltpu.make_async_copy(wo_ref, wo_s, sem_o).wait()
        colmean = acc_ref[...] * inv_l
        meanv = jax.lax.dot_general(
            colmean, wv_s[...], (((1,), (1,)), ((), ())),
            preferred_element_type=jnp.float32) + bv_ref[...]
        heads = jnp.concatenate(
            [meanv[:, h * dk:(h + 1) * dk] for h in range(_H)], axis=0)
        tiled = jnp.concatenate([heads] * (d // dk), axis=1)
        gi = jax.lax.broadcasted_iota(jnp.int32, (_NG, _H), 0)
        hi = jax.lax.broadcasted_iota(jnp.int32, (_NG, _H), 1)
        sa = (hi == 3 * gi).astype(jnp.float32)
        sb = (hi == 3 * gi + 1).astype(jnp.float32)
        sc = (hi == 3 * gi + 2).astype(jnp.float32)
        dn = (((1,), (0,)), ((), ()))
        arows = jax.lax.dot_general(sa, tiled, dn,
                                    preferred_element_type=jnp.float32)
        brows = jax.lax.dot_general(sb, tiled, dn,
                                    preferred_element_type=jnp.float32)
        crows = jax.lax.dot_general(sc, tiled, dn,
                                    preferred_element_type=jnp.float32)
        lane = jax.lax.broadcasted_iota(jnp.int32, (_NG, d), 1)
        mab = jnp.where(lane < off1, arows, brows)
        mbc = jnp.where(lane < off2, brows, crows)
        ctx20 = jnp.concatenate([tiled, mab, mbc], axis=0)
        rows_ref[0:_NROWS, :] = jax.lax.dot_general(
            ctx20, wo_s[...], (((1,), (1,)), ((), ())),
            preferred_element_type=jnp.float32) + bo_ref[...]

    @pl.when(i >= _NRED)
    def _emit():
        j = i - _NRED
        rows = rows_ref[0:_NROWS, :]
        base = j * emit_rows
        rid = jax.lax.broadcasted_iota(jnp.int32, (emit_rows, _NROWS), 0) + base
        kid = jax.lax.broadcasted_iota(jnp.int32, (emit_rows, _NROWS), 1)
        grp = rid // rows_per_group
        loc = rid - grp * rows_per_group
        rtype = jnp.where(
            loc < r1, 3 * grp,
            jnp.where(loc == r1, _H + grp,
                      jnp.where(loc < r2, 3 * grp + 1,
                                jnp.where(loc == r2, _H + _NG + grp,
                                          3 * grp + 2))))
        sel = (kid == rtype).astype(jnp.float32)
        out_ref[...] = jax.lax.dot_general(
            sel, rows, (((1,), (0,)), ((), ())),
            preferred_element_type=jnp.float32)


def kernel(queries, keys, values, Wq, bq, Wk, bk, Wv, bv, Wo, bo):
    b, l, d = values.shape
    dk = _DK
    vals2d = values.reshape(b * l, d)
    red_blk = (b * l) // _NRED
    emit_rows = (b * l) // _NEMIT

    stream = l * dk
    rows_per_group = 3 * stream // d
    r1, off1 = stream // d, stream % d
    r2, off2 = (2 * stream) // d, (2 * stream) % d

    out2d = pl.pallas_call(
        functools.partial(_fused_body, inv_l=1.0 / (b * l), d=d, dk=dk,
                          r1=r1, off1=off1, r2=r2, off2=off2,
                          rows_per_group=rows_per_group, emit_rows=emit_rows),
        grid=(_NRED + _NEMIT,),
        in_specs=[
            pl.BlockSpec((red_blk, d),
                         lambda i: (jnp.minimum(i, _NRED - 1), 0)),
            pl.BlockSpec(memory_space=pltpu.MemorySpace.HBM),
            pl.BlockSpec((1, d), lambda i: (0, 0)),
            pl.BlockSpec(memory_space=pltpu.MemorySpace.HBM),
            pl.BlockSpec((1, d), lambda i: (0, 0)),
        ],
        out_specs=pl.BlockSpec((emit_rows, d),
                               lambda i: (jnp.maximum(i, _NRED) - _NRED, 0)),
        out_shape=jax.ShapeDtypeStruct((b * l, d), jnp.float32),
        scratch_shapes=[pltpu.VMEM((1, d), jnp.float32),
                        pltpu.VMEM((24, d), jnp.float32),
                        pltpu.VMEM((d, d), jnp.float32),
                        pltpu.VMEM((d, d), jnp.float32),
                        pltpu.SemaphoreType.DMA,
                        pltpu.SemaphoreType.DMA],
    )(vals2d, Wv, bv.reshape(1, d), Wo, bo.reshape(1, d))

    return out2d.reshape(b, l, d)


# speedup vs baseline: 1.1745x; 1.0384x over previous
import functools

import jax
import jax.numpy as jnp
from jax.experimental import pallas as pl
from jax.experimental.pallas import tpu as pltpu

_H = 12
_DK = 64
_NG = _H // 3
_NRED = 4
_NEMIT = 4
_NROWS = _H + 2 * _NG


def _fused_body(values_ref, wv_ref, bv_ref, wo_ref, bo_ref, out_ref,
                acc_ref, rows_ref, sel_ref, *, inv_l, d, dk, r1, off1, r2,
                off2, rows_per_group, emit_rows):
    i = pl.program_id(0)

    @pl.when(i < _NRED)
    def _reduce():
        psum = jnp.sum(values_ref[...], axis=0, keepdims=True)
        prev = jnp.where(i == 0, jnp.zeros_like(psum), acc_ref[...])
        acc_ref[...] = prev + psum

    @pl.when(i == _NRED - 1)
    def _build_rows():
        colmean = acc_ref[...] * inv_l
        meanv = jax.lax.dot_general(
            colmean, wv_ref[...], (((1,), (1,)), ((), ())),
            preferred_element_type=jnp.float32) + bv_ref[...]
        heads = jnp.concatenate(
            [meanv[:, h * dk:(h + 1) * dk] for h in range(_H)], axis=0)
        tiled = jnp.concatenate([heads] * (d // dk), axis=1)
        gi = jax.lax.broadcasted_iota(jnp.int32, (_NG, _H), 0)
        hi = jax.lax.broadcasted_iota(jnp.int32, (_NG, _H), 1)
        sa = (hi == 3 * gi).astype(jnp.float32)
        sb = (hi == 3 * gi + 1).astype(jnp.float32)
        sc = (hi == 3 * gi + 2).astype(jnp.float32)
        dn = (((1,), (0,)), ((), ()))
        arows = jax.lax.dot_general(sa, tiled, dn,
                                    preferred_element_type=jnp.float32)
        brows = jax.lax.dot_general(sb, tiled, dn,
                                    preferred_element_type=jnp.float32)
        crows = jax.lax.dot_general(sc, tiled, dn,
                                    preferred_element_type=jnp.float32)
        lane = jax.lax.broadcasted_iota(jnp.int32, (_NG, d), 1)
        mab = jnp.where(lane < off1, arows, brows)
        mbc = jnp.where(lane < off2, brows, crows)
        zpad = jnp.zeros((3, d), jnp.float32)
        pieces = []
        for g in range(_NG):
            pieces += [tiled[3 * g:3 * g + 1], mab[g:g + 1],
                       tiled[3 * g + 1:3 * g + 2], mbc[g:g + 1],
                       tiled[3 * g + 2:3 * g + 3], zpad]
        ctx32 = jnp.concatenate(pieces, axis=0)
        rows_ref[...] = jax.lax.dot_general(
            ctx32, wo_ref[...], (((1,), (1,)), ((), ())),
            preferred_element_type=jnp.float32) + bo_ref[...]
        rid = jax.lax.broadcasted_iota(jnp.int32, (rows_per_group, 8), 0)
        kid = jax.lax.broadcasted_iota(jnp.int32, (rows_per_group, 8), 1)
        rtype = jnp.where(
            rid < r1, 0,
            jnp.where(rid == r1, 1,
                      jnp.where(rid < r2, 2,
                                jnp.where(rid == r2, 3, 4))))
        sel_ref[...] = (kid == rtype).astype(jnp.float32)

    @pl.when(i >= _NRED)
    def _emit():
        g = i - _NRED
        rows8 = rows_ref[pl.ds(8 * g, 8), :]
        out_ref[...] = jax.lax.dot_general(
            sel_ref[...], rows8, (((1,), (0,)), ((), ())),
            preferred_element_type=jnp.float32)


def kernel(queries, keys, values, Wq, bq, Wk, bk, Wv, bv, Wo, bo):
    b, l, d = values.shape
    dk = _DK
    vals2d = values.reshape(b * l, d)
    red_blk = (b * l) // _NRED
    emit_rows = (b * l) // _NEMIT

    stream = l * dk
    rows_per_group = 3 * stream // d
    r1, off1 = stream // d, stream % d
    r2, off2 = (2 * stream) // d, (2 * stream) % d

    out2d = pl.pallas_call(
        functools.partial(_fused_body, inv_l=1.0 / (b * l), d=d, dk=dk,
                          r1=r1, off1=off1, r2=r2, off2=off2,
                          rows_per_group=rows_per_group, emit_rows=emit_rows),
        grid=(_NRED + _NEMIT,),
        in_specs=[
            pl.BlockSpec((red_blk, d),
                         lambda i: (jnp.minimum(i, _NRED - 1), 0)),
            pl.BlockSpec((d, d), lambda i: (0, 0)),
            pl.BlockSpec((1, d), lambda i: (0, 0)),
            pl.BlockSpec((d, d), lambda i: (0, 0)),
            pl.BlockSpec((1, d), lambda i: (0, 0)),
        ],
        out_specs=pl.BlockSpec((emit_rows, d),
                               lambda i: (jnp.maximum(i, _NRED) - _NRED, 0)),
        out_shape=jax.ShapeDtypeStruct((b * l, d), jnp.float32),
        scratch_shapes=[pltpu.VMEM((1, d), jnp.float32),
                        pltpu.VMEM((32, d), jnp.float32),
                        pltpu.VMEM((rows_per_group, 8), jnp.float32)],
    )(vals2d, Wv, bv.reshape(1, d), Wo, bo.reshape(1, d))

    return out2d.reshape(b, l, d)
